# Initial kernel scaffold; baseline (speedup 1.0000x reference)
import jax
import jax.numpy as jnp
from jax.experimental import pallas as pl


def kernel(obs_his, cobs, pan_fut, cpan):
    raise NotImplementedError("write your pallas kernel here")


# speedup vs baseline: 1.6694x; 1.6694x over previous
import functools

import jax
import jax.numpy as jnp
from jax import lax
from jax.experimental import pallas as pl
from jax.experimental.pallas import tpu as pltpu
from jax.experimental.pallas import tpu_sc as plsc

_K = 8
_N = 512
_M = 65536
_CHUNK = 2048
_NCHUNK = _M // _CHUNK
_PAD = 128
_W = _PAD + _CHUNK
_INF = float("inf")
_BIGI = 2**30


def _topk_body(cobs_ref, cpt_ref, idx_out_ref, d_scr, i_scr):
    qx = cobs_ref[:, 0:1]
    qy = cobs_ref[:, 1:2]
    d_scr[:, 0:_PAD] = jnp.full((_N, _PAD), _INF, jnp.float32)
    i_scr[:, 0:_PAD] = jnp.full((_N, _PAD), _BIGI, jnp.int32)

    def chunk_step(c, carry):
        px = cpt_ref[0:1, pl.ds(c * _CHUNK, _CHUNK)]
        py = cpt_ref[1:2, pl.ds(c * _CHUNK, _CHUNK)]
        dx = qx - px
        dy = qy - py
        d_scr[:, _PAD:] = dx * dx + dy * dy
        i_scr[:, _PAD:] = (
            lax.broadcasted_iota(jnp.int32, (_N, _CHUNK), 1) + c * _CHUNK
        )
        wins_d = []
        wins_i = []
        for _ in range(_K):
            d = d_scr[...]
            ii = i_scr[...]
            m = jnp.min(d, axis=1, keepdims=True)
            sel = jnp.min(jnp.where(d == m, ii, _BIGI), axis=1, keepdims=True)
            d_scr[...] = jnp.where(ii == sel, _INF, d)
            wins_d.append(m)
            wins_i.append(sel)
        d_scr[:, 0:_K] = jnp.concatenate(wins_d, axis=1)
        i_scr[:, 0:_K] = jnp.concatenate(wins_i, axis=1)
        return carry

    lax.fori_loop(0, _NCHUNK, chunk_step, None)
    idx_out_ref[...] = i_scr[:, 0:_K]


def _topk_call(cobs, cpt, interpret=False):
    return pl.pallas_call(
        _topk_body,
        out_shape=jax.ShapeDtypeStruct((_N, _K), jnp.int32),
        scratch_shapes=[
            pltpu.VMEM((_N, _W), jnp.float32),
            pltpu.VMEM((_N, _W), jnp.int32),
        ],
        interpret=interpret,
    )(cobs, cpt)


_NW = 32
_RPW = 2048
_GCH = 128


def _sc_gather_body(pan_hbm, fidx_hbm, cpan_hbm, pan_out, cpan_out,
                    idx_v, rows_v, idx2_v, crows_v, sem):
    wid = lax.axis_index("s") * 2 + lax.axis_index("c")
    t = wid // 2
    half = wid % 2

    pltpu.sync_copy(fidx_hbm.at[pl.ds(half * 16, 16)], idx_v)
    off = t * _M
    for r in range(16):
        for l in range(_GCH // 16):
            sl = pl.ds(l * 16, 16)
            idx_v[r, sl] = idx_v[r, sl] + off
    copies = []
    for r in range(16):
        copies.append(
            pltpu.async_copy(
                pan_hbm.at[idx_v.at[r]], rows_v.at[pl.ds(r * _GCH, _GCH)], sem
            )
        )
    for cp in copies:
        cp.wait()
    pltpu.sync_copy(rows_v, pan_out.at[pl.ds(wid * _RPW, _RPW)])

    pltpu.sync_copy(fidx_hbm.at[pl.ds(wid, 1)], idx2_v)
    pltpu.async_copy(cpan_hbm.at[idx2_v.at[0]], crows_v, sem).wait()
    pltpu.sync_copy(crows_v, cpan_out.at[pl.ds(wid * _GCH, _GCH)])


_CPAD = 16


def _sc_gather(pan_flat, fidx, cpan):
    L = pan_flat.shape[1]
    kern = functools.partial(
        pl.kernel,
        mesh=plsc.VectorSubcoreMesh(core_axis_name="c", subcore_axis_name="s"),
        out_type=[
            jax.ShapeDtypeStruct((_NW * _RPW, L), jnp.float32),
            jax.ShapeDtypeStruct((_N * _K, _CPAD), jnp.float32),
        ],
        scratch_types=[
            pltpu.VMEM((16, _GCH), jnp.int32),
            pltpu.VMEM((_RPW, L), jnp.float32),
            pltpu.VMEM((1, _GCH), jnp.int32),
            pltpu.VMEM((_GCH, _CPAD), jnp.float32),
            pltpu.SemaphoreType.DMA,
        ],
        compiler_params=pltpu.CompilerParams(use_tc_tiling_on_sc=False),
    )(_sc_gather_body)
    return kern(pan_flat, fidx, cpan)


def kernel(obs_his, cobs, pan_fut, cpan):
    B, C, N, L = obs_his.shape
    pan_fut = pan_fut.reshape(B, C, -1, L)
    cpan_flat = cpan.reshape(-1, 2)
    cpt = cpan_flat.T.reshape(2, _M)

    idx = _topk_call(cobs, cpt)
    fidx = idx.reshape(_NW, _GCH)
    pan_flat = pan_fut.reshape(B * C * _M, L)
    cpan_pad = jnp.pad(cpan_flat, ((0, 0), (0, _CPAD - 2)))

    pan_rows, cpan_rows = _sc_gather(pan_flat, fidx, cpan_pad)
    pan_k = pan_rows.reshape(B, C, N, _K, L)
    cpan_k = cpan_rows[:, :2].reshape(N, _K, 2)
    return pan_k, cpan_k


# speedup vs baseline: 1.7571x; 1.0525x over previous
import functools

import jax
import jax.numpy as jnp
from jax import lax
from jax.experimental import pallas as pl
from jax.experimental.pallas import tpu as pltpu
from jax.experimental.pallas import tpu_sc as plsc

_K = 8
_N = 512
_M = 65536
_CHUNK = 2048
_NCHUNK = _M // _CHUNK
_PAD = 128
_W = _PAD + _CHUNK
_INF = float("inf")
_BIGF = float(2**24)


def _topk_body(cobs_ref, cpt_ref, idx_out_ref, d_scr, i_scr):
    qx = cobs_ref[:, 0:1]
    qy = cobs_ref[:, 1:2]
    d_scr[:, 0:_PAD] = jnp.full((_N, _PAD), _INF, jnp.float32)
    i_scr[:, 0:_PAD] = jnp.full((_N, _PAD), _BIGF, jnp.float32)

    def chunk_step(c, carry):
        px = cpt_ref[0:1, pl.ds(c * _CHUNK, _CHUNK)]
        py = cpt_ref[1:2, pl.ds(c * _CHUNK, _CHUNK)]
        dx = qx - px
        dy = qy - py
        d_scr[:, _PAD:] = dx * dx + dy * dy
        i_scr[:, _PAD:] = (
            lax.broadcasted_iota(jnp.int32, (_N, _CHUNK), 1).astype(jnp.float32)
            + (c * _CHUNK).astype(jnp.float32)
        )
        wins_d = []
        wins_i = []
        for _ in range(_K):
            d = d_scr[...]
            ii = i_scr[...]
            m = jnp.min(d, axis=1, keepdims=True)
            sel = jnp.min(jnp.where(d == m, ii, _BIGF), axis=1, keepdims=True)
            d_scr[...] = jnp.where(ii == sel, _INF, d)
            wins_d.append(m)
            wins_i.append(sel)
        d_scr[:, 0:_K] = jnp.concatenate(wins_d, axis=1)
        i_scr[:, 0:_K] = jnp.concatenate(wins_i, axis=1)
        return carry

    lax.fori_loop(0, _NCHUNK, chunk_step, None)
    idx_out_ref[...] = i_scr[:, 0:_K].astype(jnp.int32)


def _topk_call(cobs, cpt, interpret=False):
    return pl.pallas_call(
        _topk_body,
        out_shape=jax.ShapeDtypeStruct((_N, _K), jnp.int32),
        scratch_shapes=[
            pltpu.VMEM((_N, _W), jnp.float32),
            pltpu.VMEM((_N, _W), jnp.float32),
        ],
        interpret=interpret,
    )(cobs, cpt)


_NW = 32
_RPW = 2048
_GCH = 128


def _sc_gather_body(pan_hbm, fidx_hbm, cpan_hbm, pan_out, cpan_out,
                    idx_v, rows_v, idx2_v, crows_v, sem):
    wid = lax.axis_index("s") * 2 + lax.axis_index("c")
    t = wid // 2
    half = wid % 2

    pltpu.sync_copy(fidx_hbm.at[pl.ds(half * 16, 16)], idx_v)
    off = t * _M
    for r in range(16):
        for l in range(_GCH // 16):
            sl = pl.ds(l * 16, 16)
            idx_v[r, sl] = idx_v[r, sl] + off
    copies = []
    for r in range(16):
        copies.append(
            pltpu.async_copy(
                pan_hbm.at[idx_v.at[r]], rows_v.at[pl.ds(r * _GCH, _GCH)], sem
            )
        )
    for cp in copies:
        cp.wait()
    pltpu.sync_copy(rows_v, pan_out.at[pl.ds(wid * _RPW, _RPW)])

    pltpu.sync_copy(fidx_hbm.at[pl.ds(wid, 1)], idx2_v)
    pltpu.async_copy(cpan_hbm.at[idx2_v.at[0]], crows_v, sem).wait()
    pltpu.sync_copy(crows_v, cpan_out.at[pl.ds(wid * _GCH, _GCH)])


_CPAD = 16


def _sc_gather(pan_flat, fidx, cpan):
    L = pan_flat.shape[1]
    kern = functools.partial(
        pl.kernel,
        mesh=plsc.VectorSubcoreMesh(core_axis_name="c", subcore_axis_name="s"),
        out_type=[
            jax.ShapeDtypeStruct((_NW * _RPW, L), jnp.float32),
            jax.ShapeDtypeStruct((_N * _K, _CPAD), jnp.float32),
        ],
        scratch_types=[
            pltpu.VMEM((16, _GCH), jnp.int32),
            pltpu.VMEM((_RPW, L), jnp.float32),
            pltpu.VMEM((1, _GCH), jnp.int32),
            pltpu.VMEM((_GCH, _CPAD), jnp.float32),
            pltpu.SemaphoreType.DMA,
        ],
        compiler_params=pltpu.CompilerParams(use_tc_tiling_on_sc=False),
    )(_sc_gather_body)
    return kern(pan_flat, fidx, cpan)


def kernel(obs_his, cobs, pan_fut, cpan):
    B, C, N, L = obs_his.shape
    pan_fut = pan_fut.reshape(B, C, -1, L)
    cpan_flat = cpan.reshape(-1, 2)
    cpt = cpan_flat.T.reshape(2, _M)

    idx = _topk_call(cobs, cpt)
    fidx = idx.reshape(_NW, _GCH)
    pan_flat = pan_fut.reshape(B * C * _M, L)
    cpan_pad = jnp.pad(cpan_flat, ((0, 0), (0, _CPAD - 2)))

    pan_rows, cpan_rows = _sc_gather(pan_flat, fidx, cpan_pad)
    pan_k = pan_rows.reshape(B, C, N, _K, L)
    cpan_k = cpan_rows[:, :2].reshape(N, _K, 2)
    return pan_k, cpan_k
